# baseline (speedup 1.0000x reference)
import jax
import jax.numpy as jnp
from jax.experimental import pallas as pl
from jax.experimental.pallas import tpu as pltpu

N = 10000
EPS = 1e-5


def _gcn(x, row, col, w, W, b, n):
    h = x @ W
    deg = jax.ops.segment_sum(w, row, num_segments=n)
    dinv = jnp.where(deg > 0, jax.lax.rsqrt(jnp.maximum(deg, 1e-12)), 0.0)
    norm = dinv[row] * w * dinv[col]
    out = jax.ops.segment_sum(h[row] * norm[:, None], col, num_segments=n)
    return out + b


def _bn(x, g, b):
    m = x.mean(axis=0)
    v = x.var(axis=0)
    return (x - m) / jnp.sqrt(v + EPS) * g + b


def _lstm_step(x, Wih, bih, bhh):
    gates = x @ Wih.T + bih + bhh
    i, f, g, o = jnp.split(gates, 4, axis=-1)
    c = jax.nn.sigmoid(i) * jnp.tanh(g)
    return jax.nn.sigmoid(o) * jnp.tanh(c)


def _fc_head_kernel(z_ref, w1_ref, b1_ref, w2_ref, b2_ref, o_ref):
    z = z_ref[...]
    h = jax.nn.relu(
        jax.lax.dot_general(z, w1_ref[...], (((1,), (1,)), ((), ())),
                            preferred_element_type=jnp.float32,
                            precision=jax.lax.Precision.HIGHEST)
        + b1_ref[...][None, :])
    y = (jax.lax.dot_general(h, w2_ref[...], (((1,), (1,)), ((), ())),
                             preferred_element_type=jnp.float32,
                             precision=jax.lax.Precision.HIGHEST)
         + b2_ref[...][None, :])
    m = jnp.max(y, axis=1, keepdims=True)
    s = jnp.log(jnp.sum(jnp.exp(y - m), axis=1, keepdims=True))
    o_ref[...] = y - m - s


def kernel(x, edge_index, edge_attr, conv1_W, conv1_b, conv2_W, conv2_b,
           bn1_g, bn1_b, bn2_g, bn2_b,
           l1f_Wih, l1f_bih, l1f_bhh, l1b_Wih, l1b_bih, l1b_bhh,
           l2f_Wih, l2f_bih, l2f_bhh, l2b_Wih, l2b_bih, l2b_bhh,
           fc1_W, fc1_b, fc2_W, fc2_b):
    n = x.shape[0]
    loop = jnp.arange(n, dtype=edge_index.dtype)
    row = jnp.concatenate([edge_index[0], loop])
    col = jnp.concatenate([edge_index[1], loop])
    w = jnp.concatenate([edge_attr, jnp.ones((n,), jnp.float32)])
    skip = x
    h1 = jax.nn.relu(_gcn(x, row, col, w, conv1_W, conv1_b, n))
    h1 = _bn(h1, bn1_g, bn1_b)
    h2 = jax.nn.relu(_gcn(h1, row, col, w, conv2_W, conv2_b, n))
    h2 = _bn(h2, bn2_g, bn2_b)
    cat = jnp.concatenate([skip, h1, h2], axis=1)
    hf1 = _lstm_step(cat, l1f_Wih, l1f_bih, l1f_bhh)
    hb1 = _lstm_step(cat, l1b_Wih, l1b_bih, l1b_bhh)
    out1 = jnp.concatenate([hf1, hb1], axis=1)
    hf2 = _lstm_step(out1, l2f_Wih, l2f_bih, l2f_bhh)
    hb2 = _lstm_step(out1, l2b_Wih, l2b_bih, l2b_bhh)
    out2 = jnp.concatenate([hf2, hb2], axis=1)
    z = jnp.concatenate([out1, out2, skip], axis=1)
    blk = 2000
    return pl.pallas_call(
        _fc_head_kernel,
        grid=(n // blk,),
        in_specs=[
            pl.BlockSpec((blk, z.shape[1]), lambda i: (i, 0)),
            pl.BlockSpec(fc1_W.shape, lambda i: (0, 0)),
            pl.BlockSpec(fc1_b.shape, lambda i: (0,)),
            pl.BlockSpec(fc2_W.shape, lambda i: (0, 0)),
            pl.BlockSpec(fc2_b.shape, lambda i: (0,)),
        ],
        out_specs=pl.BlockSpec((blk, fc2_W.shape[0]), lambda i: (i, 0)),
        out_shape=jax.ShapeDtypeStruct((n, fc2_W.shape[0]), jnp.float32),
    )(z, fc1_W, fc1_b, fc2_W, fc2_b)


# speedup vs baseline: 9.9687x; 9.9687x over previous
import dataclasses
import functools

import jax
import jax.numpy as jnp
from jax import lax
from jax.experimental import pallas as pl
from jax.experimental.pallas import tpu as pltpu
from jax.experimental.pallas import tpu_sc as plsc

_N = 10000
_E = 320000
_NW = 32
_EPW = _E // _NW
_EC = 2000
_NCH = _E // _EC
_HID = 64

_HIGH = lax.Precision.HIGHEST


def _sc_mesh():
    return plsc.VectorSubcoreMesh(core_axis_name="c", subcore_axis_name="s",
                                  num_cores=2, num_subcores=16)


def _sc_params():
    cp = pltpu.CompilerParams()
    if "needs_layout_passes" in pltpu.CompilerParams.__dataclass_fields__:
        cp = dataclasses.replace(cp, needs_layout_passes=False)
    return cp


def _sc_deg(row, w):
    @functools.partial(
        pl.kernel,
        out_type=jax.ShapeDtypeStruct((_NW * _N,), jnp.float32),
        mesh=_sc_mesh(),
        compiler_params=_sc_params(),
        scratch_types=[
            pltpu.VMEM((_EPW,), jnp.int32),
            pltpu.VMEM((_EPW,), jnp.float32),
            pltpu.VMEM((_N,), jnp.float32),
        ],
    )
    def k(row_hbm, w_hbm, out_hbm, row_v, w_v, deg_v):
        wid = lax.axis_index("s") * 2 + lax.axis_index("c")
        base = wid * _EPW
        zeros = jnp.zeros((16,), jnp.float32)

        @pl.loop(0, _N, step=16)
        def _(i):
            deg_v[pl.ds(i, 16)] = zeros

        pltpu.sync_copy(row_hbm.at[pl.ds(base, _EPW)], row_v)
        pltpu.sync_copy(w_hbm.at[pl.ds(base, _EPW)], w_v)

        @pl.loop(0, _EPW, step=16)
        def _(i):
            idx = row_v[pl.ds(i, 16)]
            plsc.addupdate_scatter(deg_v, [idx], w_v[pl.ds(i, 16)])

        pltpu.sync_copy(deg_v, out_hbm.at[pl.ds(wid * _N, _N)])

    return k(row, w).reshape(_NW, _N)


def _sc_conv(hsT, row, col, w):
    @functools.partial(
        pl.kernel,
        out_type=jax.ShapeDtypeStruct((_HID * _N,), jnp.float32),
        mesh=_sc_mesh(),
        compiler_params=_sc_params(),
        scratch_types=[
            pltpu.VMEM((2 * _N,), jnp.float32),
            pltpu.VMEM((2 * _N,), jnp.float32),
            pltpu.VMEM((_EC,), jnp.int32),
            pltpu.VMEM((_EC,), jnp.int32),
            pltpu.VMEM((_EC,), jnp.int32),
            pltpu.VMEM((_EC,), jnp.int32),
            pltpu.VMEM((_EC,), jnp.float32),
            pltpu.VMEM((_EC,), jnp.float32),
            pltpu.SemaphoreType.DMA,
            pltpu.SemaphoreType.DMA,
        ],
    )
    def k(h_hbm, row_hbm, col_hbm, w_hbm, out_hbm,
          ht_v, acc_v, row_v0, row_v1, col_v0, col_v1, w_v0, w_v1, sem0, sem1):
        wid = lax.axis_index("s") * 2 + lax.axis_index("c")
        f0 = wid * 2
        bufs = ((row_v0, col_v0, w_v0, sem0), (row_v1, col_v1, w_v1, sem1))

        pltpu.sync_copy(h_hbm.at[pl.ds(f0 * _N, 2 * _N)], ht_v)

        zeros = jnp.zeros((16,), jnp.float32)

        @pl.loop(0, 2 * _N, step=16)
        def _(i):
            acc_v[pl.ds(i, 16)] = zeros

        def start(ch, b):
            off = ch * _EC
            rv, cv, wv_, sem = bufs[b]
            pltpu.async_copy(row_hbm.at[pl.ds(off, _EC)], rv, sem)
            pltpu.async_copy(col_hbm.at[pl.ds(off, _EC)], cv, sem)
            pltpu.async_copy(w_hbm.at[pl.ds(off, _EC)], wv_, sem)

        def wait(b):
            rv, cv, wv_, sem = bufs[b]
            pltpu.make_async_copy(row_hbm.at[pl.ds(0, _EC)], rv, sem).wait()
            pltpu.make_async_copy(col_hbm.at[pl.ds(0, _EC)], cv, sem).wait()
            pltpu.make_async_copy(w_hbm.at[pl.ds(0, _EC)], wv_, sem).wait()

        def process(b):
            rv, cv, wv_, _sem = bufs[b]

            @pl.loop(0, _EC, step=16)
            def _(i):
                r = rv[pl.ds(i, 16)]
                c = cv[pl.ds(i, 16)]
                wv = wv_[pl.ds(i, 16)]
                g0 = plsc.load_gather(ht_v, [r])
                plsc.addupdate_scatter(acc_v, [c], g0 * wv)
                r1 = r + _N
                c1 = c + _N
                g1 = plsc.load_gather(ht_v, [r1])
                plsc.addupdate_scatter(acc_v, [c1], g1 * wv)

        start(0, 0)
        start(1, 1)

        @pl.loop(0, _NCH, step=2)
        def _(ch):
            wait(0)
            process(0)

            @pl.when(ch + 2 < _NCH)
            def _():
                start(ch + 2, 0)

            wait(1)
            process(1)

            @pl.when(ch + 3 < _NCH)
            def _():
                start(ch + 3, 1)

        pltpu.sync_copy(acc_v, out_hbm.at[pl.ds(f0 * _N, 2 * _N)])

    return k(hsT.reshape(-1), row, col, w).reshape(_HID, _N)


def _dotT(a, b):
    return lax.dot_general(a, b, (((0,), (0,)), ((), ())),
                           preferred_element_type=jnp.float32,
                           precision=_HIGH)


def _mm(w, b):
    return lax.dot_general(w, b, (((1,), (0,)), ((), ())),
                           preferred_element_type=jnp.float32,
                           precision=_HIGH)


def _tc_pre(x, conv1_W, deg_parts):
    def body(x_ref, w_ref, dp_ref, xT_ref, h1s_ref, dinv_ref):
        xT = x_ref[...].T
        xT_ref[...] = xT
        deg = jnp.sum(dp_ref[...], axis=0, keepdims=True) + 1.0
        dinv = jnp.where(deg > 0, lax.rsqrt(jnp.maximum(deg, 1e-12)), 0.0)
        dinv_ref[...] = dinv
        h1s_ref[...] = _dotT(w_ref[...], xT) * dinv

    return pl.pallas_call(
        body,
        out_shape=(
            jax.ShapeDtypeStruct((128, _N), jnp.float32),
            jax.ShapeDtypeStruct((_HID, _N), jnp.float32),
            jax.ShapeDtypeStruct((1, _N), jnp.float32),
        ),
    )(x, conv1_W, deg_parts)


def _bn_T(h, g_col, b_col):
    m = jnp.mean(h, axis=1, keepdims=True)
    v = jnp.mean((h - m) ** 2, axis=1, keepdims=True)
    return (h - m) / jnp.sqrt(v + 1e-5) * g_col + b_col


def _tc_mid(acc1, h1s, dinv, b1_col, g1_col, bb1_col, conv2_W):
    def body(acc_ref, h1s_ref, dinv_ref, b1_ref, g1_ref, bb1_ref, w2_ref,
             h1T_ref, h2s_ref):
        dinv = dinv_ref[...]
        pre = dinv * (acc_ref[...] + h1s_ref[...]) + b1_ref[...]
        h1 = _bn_T(jax.nn.relu(pre), g1_ref[...], bb1_ref[...])
        h1T_ref[...] = h1
        h2s_ref[...] = _dotT(w2_ref[...], h1) * dinv

    return pl.pallas_call(
        body,
        out_shape=(
            jax.ShapeDtypeStruct((_HID, _N), jnp.float32),
            jax.ShapeDtypeStruct((_HID, _N), jnp.float32),
        ),
    )(acc1, h1s, dinv, b1_col, g1_col, bb1_col, conv2_W)


def _tc_post(acc2, h2s, dinv, b2_col, g2_col, bb2_col):
    def body(acc_ref, h2s_ref, dinv_ref, b2_ref, g2_ref, bb2_ref, h2T_ref):
        pre = dinv_ref[...] * (acc_ref[...] + h2s_ref[...]) + b2_ref[...]
        h2T_ref[...] = _bn_T(jax.nn.relu(pre), g2_ref[...], bb2_ref[...])

    return pl.pallas_call(
        body,
        out_shape=jax.ShapeDtypeStruct((_HID, _N), jnp.float32),
    )(acc2, h2s, dinv, b2_col, g2_col, bb2_col)


def _lstm_gates(gates):
    i = gates[0:_HID]
    f = gates[_HID:2 * _HID]
    g = gates[2 * _HID:3 * _HID]
    o = gates[3 * _HID:4 * _HID]
    del f
    c = jax.nn.sigmoid(i) * jnp.tanh(g)
    return jax.nn.sigmoid(o) * jnp.tanh(c)


def _tc_lstm1(Wx, Wh1, Wh2, b_col, xT, h1T, h2T):
    def body(wx_ref, w1_ref, w2_ref, b_ref, xT_ref, h1_ref, h2_ref, o_ref):
        gates = (_mm(wx_ref[...], xT_ref[...])
                 + _mm(w1_ref[...], h1_ref[...])
                 + _mm(w2_ref[...], h2_ref[...])
                 + b_ref[...])
        o_ref[...] = _lstm_gates(gates)

    return pl.pallas_call(
        body,
        out_shape=jax.ShapeDtypeStruct((_HID, _N), jnp.float32),
    )(Wx, Wh1, Wh2, b_col, xT, h1T, h2T)


def _tc_lstm2(Wa, Wb, b_col, hfT, hbT):
    def body(wa_ref, wb_ref, b_ref, hf_ref, hb_ref, o_ref):
        gates = (_mm(wa_ref[...], hf_ref[...])
                 + _mm(wb_ref[...], hb_ref[...])
                 + b_ref[...])
        o_ref[...] = _lstm_gates(gates)

    return pl.pallas_call(
        body,
        out_shape=jax.ShapeDtypeStruct((_HID, _N), jnp.float32),
    )(Wa, Wb, b_col, hfT, hbT)


def _tc_fc(W_parts, b1_col, fc2_W, b2_col, acts):
    def body(*refs):
        w_refs = refs[:5]
        b1_ref, w2_ref, b2_ref = refs[5:8]
        act_refs = refs[8:13]
        o_ref = refs[13]
        z = b1_ref[...]
        for w_r, a_r in zip(w_refs, act_refs):
            z = z + _mm(w_r[...], a_r[...])
        z = jax.nn.relu(z)
        y = _mm(w2_ref[...], z) + b2_ref[...]
        m = jnp.max(y, axis=0, keepdims=True)
        ls = y - m - jnp.log(jnp.sum(jnp.exp(y - m), axis=0, keepdims=True))
        o_ref[...] = ls.T

    return pl.pallas_call(
        body,
        out_shape=jax.ShapeDtypeStruct((_N, 10), jnp.float32),
    )(*W_parts, b1_col, fc2_W, b2_col, *acts)


def kernel(x, edge_index, edge_attr, conv1_W, conv1_b, conv2_W, conv2_b,
           bn1_g, bn1_b, bn2_g, bn2_b,
           l1f_Wih, l1f_bih, l1f_bhh, l1b_Wih, l1b_bih, l1b_bhh,
           l2f_Wih, l2f_bih, l2f_bhh, l2b_Wih, l2b_bih, l2b_bhh,
           fc1_W, fc1_b, fc2_W, fc2_b):
    row = edge_index[0]
    col = edge_index[1]

    col_of = lambda v: v.reshape(-1, 1)

    deg_parts = _sc_deg(row, edge_attr)
    xT, h1sT, dinv = _tc_pre(x, conv1_W, deg_parts)

    acc1 = _sc_conv(h1sT, row, col, edge_attr)
    h1T, h2sT = _tc_mid(acc1, h1sT, dinv, col_of(conv1_b), col_of(bn1_g),
                        col_of(bn1_b), conv2_W)

    acc2 = _sc_conv(h2sT, row, col, edge_attr)
    h2T = _tc_post(acc2, h2sT, dinv, col_of(conv2_b), col_of(bn2_g),
                   col_of(bn2_b))

    def split_l1(W):
        return W[:, :128], W[:, 128:192], W[:, 192:256]

    hf1 = _tc_lstm1(*split_l1(l1f_Wih), col_of(l1f_bih + l1f_bhh), xT, h1T, h2T)
    hb1 = _tc_lstm1(*split_l1(l1b_Wih), col_of(l1b_bih + l1b_bhh), xT, h1T, h2T)

    hf2 = _tc_lstm2(l2f_Wih[:, :64], l2f_Wih[:, 64:],
                    col_of(l2f_bih + l2f_bhh), hf1, hb1)
    hb2 = _tc_lstm2(l2b_Wih[:, :64], l2b_Wih[:, 64:],
                    col_of(l2b_bih + l2b_bhh), hf1, hb1)

    W_parts = (fc1_W[:, :64], fc1_W[:, 64:128], fc1_W[:, 128:192],
               fc1_W[:, 192:256], fc1_W[:, 256:384])
    acts = (hf1, hb1, hf2, hb2, xT)
    return _tc_fc(W_parts, col_of(fc1_b), fc2_W, col_of(fc2_b), acts)


# speedup vs baseline: 10.1481x; 1.0180x over previous
import dataclasses
import functools

import jax
import jax.numpy as jnp
from jax import lax
from jax.experimental import pallas as pl
from jax.experimental.pallas import tpu as pltpu
from jax.experimental.pallas import tpu_sc as plsc

_N = 10000
_E = 320000
_NW = 32
_EPW = _E // _NW
_EC = 2000
_NCH = _E // _EC
_HID = 64

_HIGH = lax.Precision.HIGHEST


def _sc_mesh():
    return plsc.VectorSubcoreMesh(core_axis_name="c", subcore_axis_name="s",
                                  num_cores=2, num_subcores=16)


def _sc_params():
    cp = pltpu.CompilerParams()
    if "needs_layout_passes" in pltpu.CompilerParams.__dataclass_fields__:
        cp = dataclasses.replace(cp, needs_layout_passes=False)
    return cp


def _sc_deg(row, w):
    @functools.partial(
        pl.kernel,
        out_type=jax.ShapeDtypeStruct((_NW * _N,), jnp.float32),
        mesh=_sc_mesh(),
        compiler_params=_sc_params(),
        scratch_types=[
            pltpu.VMEM((_EPW,), jnp.int32),
            pltpu.VMEM((_EPW,), jnp.float32),
            pltpu.VMEM((_N,), jnp.float32),
        ],
    )
    def k(row_hbm, w_hbm, out_hbm, row_v, w_v, deg_v):
        wid = lax.axis_index("s") * 2 + lax.axis_index("c")
        base = wid * _EPW
        zeros = jnp.zeros((16,), jnp.float32)

        @pl.loop(0, _N, step=16)
        def _(i):
            deg_v[pl.ds(i, 16)] = zeros

        pltpu.sync_copy(row_hbm.at[pl.ds(base, _EPW)], row_v)
        pltpu.sync_copy(w_hbm.at[pl.ds(base, _EPW)], w_v)

        @pl.loop(0, _EPW, step=16)
        def _(i):
            idx = row_v[pl.ds(i, 16)]
            plsc.addupdate_scatter(deg_v, [idx], w_v[pl.ds(i, 16)])

        pltpu.sync_copy(deg_v, out_hbm.at[pl.ds(wid * _N, _N)])

    return k(row, w).reshape(_NW, _N)


def _sc_conv(hsT, row, col, w):
    @functools.partial(
        pl.kernel,
        out_type=jax.ShapeDtypeStruct((_HID * _N,), jnp.float32),
        mesh=_sc_mesh(),
        compiler_params=_sc_params(),
        scratch_types=[
            pltpu.VMEM((2 * _N,), jnp.float32),
            pltpu.VMEM((2 * _N,), jnp.float32),
            pltpu.VMEM((_EC,), jnp.int32),
            pltpu.VMEM((_EC,), jnp.int32),
            pltpu.VMEM((_EC,), jnp.int32),
            pltpu.VMEM((_EC,), jnp.int32),
            pltpu.VMEM((_EC,), jnp.float32),
            pltpu.VMEM((_EC,), jnp.float32),
            pltpu.SemaphoreType.DMA,
            pltpu.SemaphoreType.DMA,
        ],
    )
    def k(h_hbm, row_hbm, col_hbm, w_hbm, out_hbm,
          ht_v, acc_v, row_v0, row_v1, col_v0, col_v1, w_v0, w_v1, sem0, sem1):
        wid = lax.axis_index("s") * 2 + lax.axis_index("c")
        f0 = wid * 2
        bufs = ((row_v0, col_v0, w_v0, sem0), (row_v1, col_v1, w_v1, sem1))

        pltpu.sync_copy(h_hbm.at[pl.ds(f0 * _N, 2 * _N)], ht_v)

        zeros = jnp.zeros((16,), jnp.float32)

        @pl.loop(0, 2 * _N, step=16)
        def _(i):
            acc_v[pl.ds(i, 16)] = zeros

        def start(ch, b):
            off = ch * _EC
            rv, cv, wv_, sem = bufs[b]
            pltpu.async_copy(row_hbm.at[pl.ds(off, _EC)], rv, sem)
            pltpu.async_copy(col_hbm.at[pl.ds(off, _EC)], cv, sem)
            pltpu.async_copy(w_hbm.at[pl.ds(off, _EC)], wv_, sem)

        def wait(b):
            rv, cv, wv_, sem = bufs[b]
            pltpu.make_async_copy(row_hbm.at[pl.ds(0, _EC)], rv, sem).wait()
            pltpu.make_async_copy(col_hbm.at[pl.ds(0, _EC)], cv, sem).wait()
            pltpu.make_async_copy(w_hbm.at[pl.ds(0, _EC)], wv_, sem).wait()

        def process(b):
            rv, cv, wv_, _sem = bufs[b]

            @pl.loop(0, _EC, step=16, unroll=4)
            def _(i):
                r = rv[pl.ds(i, 16)]
                c = cv[pl.ds(i, 16)]
                wv = wv_[pl.ds(i, 16)]
                g0 = plsc.load_gather(ht_v, [r])
                plsc.addupdate_scatter(acc_v, [c], g0 * wv)
                r1 = r + _N
                c1 = c + _N
                g1 = plsc.load_gather(ht_v, [r1])
                plsc.addupdate_scatter(acc_v, [c1], g1 * wv)

        start(0, 0)
        start(1, 1)

        @pl.loop(0, _NCH, step=2)
        def _(ch):
            wait(0)
            process(0)

            @pl.when(ch + 2 < _NCH)
            def _():
                start(ch + 2, 0)

            wait(1)
            process(1)

            @pl.when(ch + 3 < _NCH)
            def _():
                start(ch + 3, 1)

        pltpu.sync_copy(acc_v, out_hbm.at[pl.ds(f0 * _N, 2 * _N)])

    return k(hsT.reshape(-1), row, col, w).reshape(_HID, _N)


def _dotT(a, b):
    return lax.dot_general(a, b, (((0,), (0,)), ((), ())),
                           preferred_element_type=jnp.float32,
                           precision=_HIGH)


def _mm(w, b):
    return lax.dot_general(w, b, (((1,), (0,)), ((), ())),
                           preferred_element_type=jnp.float32,
                           precision=_HIGH)


def _tc_pre(x, conv1_W, deg_parts):
    def body(x_ref, w_ref, dp_ref, xT_ref, h1s_ref, dinv_ref):
        xT = x_ref[...].T
        xT_ref[...] = xT
        deg = jnp.sum(dp_ref[...], axis=0, keepdims=True) + 1.0
        dinv = jnp.where(deg > 0, lax.rsqrt(jnp.maximum(deg, 1e-12)), 0.0)
        dinv_ref[...] = dinv
        h1s_ref[...] = _dotT(w_ref[...], xT) * dinv

    return pl.pallas_call(
        body,
        out_shape=(
            jax.ShapeDtypeStruct((128, _N), jnp.float32),
            jax.ShapeDtypeStruct((_HID, _N), jnp.float32),
            jax.ShapeDtypeStruct((1, _N), jnp.float32),
        ),
    )(x, conv1_W, deg_parts)


def _bn_T(h, g_col, b_col):
    m = jnp.mean(h, axis=1, keepdims=True)
    v = jnp.mean((h - m) ** 2, axis=1, keepdims=True)
    return (h - m) / jnp.sqrt(v + 1e-5) * g_col + b_col


def _tc_mid(acc1, h1s, dinv, b1_col, g1_col, bb1_col, conv2_W):
    def body(acc_ref, h1s_ref, dinv_ref, b1_ref, g1_ref, bb1_ref, w2_ref,
             h1T_ref, h2s_ref):
        dinv = dinv_ref[...]
        pre = dinv * (acc_ref[...] + h1s_ref[...]) + b1_ref[...]
        h1 = _bn_T(jax.nn.relu(pre), g1_ref[...], bb1_ref[...])
        h1T_ref[...] = h1
        h2s_ref[...] = _dotT(w2_ref[...], h1) * dinv

    return pl.pallas_call(
        body,
        out_shape=(
            jax.ShapeDtypeStruct((_HID, _N), jnp.float32),
            jax.ShapeDtypeStruct((_HID, _N), jnp.float32),
        ),
    )(acc1, h1s, dinv, b1_col, g1_col, bb1_col, conv2_W)


def _tc_post(acc2, h2s, dinv, b2_col, g2_col, bb2_col):
    def body(acc_ref, h2s_ref, dinv_ref, b2_ref, g2_ref, bb2_ref, h2T_ref):
        pre = dinv_ref[...] * (acc_ref[...] + h2s_ref[...]) + b2_ref[...]
        h2T_ref[...] = _bn_T(jax.nn.relu(pre), g2_ref[...], bb2_ref[...])

    return pl.pallas_call(
        body,
        out_shape=jax.ShapeDtypeStruct((_HID, _N), jnp.float32),
    )(acc2, h2s, dinv, b2_col, g2_col, bb2_col)


def _lstm_gates(gates):
    i = gates[0:_HID]
    f = gates[_HID:2 * _HID]
    g = gates[2 * _HID:3 * _HID]
    o = gates[3 * _HID:4 * _HID]
    del f
    c = jax.nn.sigmoid(i) * jnp.tanh(g)
    return jax.nn.sigmoid(o) * jnp.tanh(c)


def _tc_lstm1(Wx, Wh1, Wh2, b_col, xT, h1T, h2T):
    def body(wx_ref, w1_ref, w2_ref, b_ref, xT_ref, h1_ref, h2_ref, o_ref):
        gates = (_mm(wx_ref[...], xT_ref[...])
                 + _mm(w1_ref[...], h1_ref[...])
                 + _mm(w2_ref[...], h2_ref[...])
                 + b_ref[...])
        o_ref[...] = _lstm_gates(gates)

    return pl.pallas_call(
        body,
        out_shape=jax.ShapeDtypeStruct((_HID, _N), jnp.float32),
    )(Wx, Wh1, Wh2, b_col, xT, h1T, h2T)


def _tc_lstm2(Wa, Wb, b_col, hfT, hbT):
    def body(wa_ref, wb_ref, b_ref, hf_ref, hb_ref, o_ref):
        gates = (_mm(wa_ref[...], hf_ref[...])
                 + _mm(wb_ref[...], hb_ref[...])
                 + b_ref[...])
        o_ref[...] = _lstm_gates(gates)

    return pl.pallas_call(
        body,
        out_shape=jax.ShapeDtypeStruct((_HID, _N), jnp.float32),
    )(Wa, Wb, b_col, hfT, hbT)


def _tc_fc(W_parts, b1_col, fc2_W, b2_col, acts):
    def body(*refs):
        w_refs = refs[:5]
        b1_ref, w2_ref, b2_ref = refs[5:8]
        act_refs = refs[8:13]
        o_ref = refs[13]
        z = b1_ref[...]
        for w_r, a_r in zip(w_refs, act_refs):
            z = z + _mm(w_r[...], a_r[...])
        z = jax.nn.relu(z)
        y = _mm(w2_ref[...], z) + b2_ref[...]
        m = jnp.max(y, axis=0, keepdims=True)
        ls = y - m - jnp.log(jnp.sum(jnp.exp(y - m), axis=0, keepdims=True))
        o_ref[...] = ls.T

    return pl.pallas_call(
        body,
        out_shape=jax.ShapeDtypeStruct((_N, 10), jnp.float32),
    )(*W_parts, b1_col, fc2_W, b2_col, *acts)


def kernel(x, edge_index, edge_attr, conv1_W, conv1_b, conv2_W, conv2_b,
           bn1_g, bn1_b, bn2_g, bn2_b,
           l1f_Wih, l1f_bih, l1f_bhh, l1b_Wih, l1b_bih, l1b_bhh,
           l2f_Wih, l2f_bih, l2f_bhh, l2b_Wih, l2b_bih, l2b_bhh,
           fc1_W, fc1_b, fc2_W, fc2_b):
    row = edge_index[0]
    col = edge_index[1]

    col_of = lambda v: v.reshape(-1, 1)

    deg_parts = _sc_deg(row, edge_attr)
    xT, h1sT, dinv = _tc_pre(x, conv1_W, deg_parts)

    acc1 = _sc_conv(h1sT, row, col, edge_attr)
    h1T, h2sT = _tc_mid(acc1, h1sT, dinv, col_of(conv1_b), col_of(bn1_g),
                        col_of(bn1_b), conv2_W)

    acc2 = _sc_conv(h2sT, row, col, edge_attr)
    h2T = _tc_post(acc2, h2sT, dinv, col_of(conv2_b), col_of(bn2_g),
                   col_of(bn2_b))

    def split_l1(W):
        return W[:, :128], W[:, 128:192], W[:, 192:256]

    hf1 = _tc_lstm1(*split_l1(l1f_Wih), col_of(l1f_bih + l1f_bhh), xT, h1T, h2T)
    hb1 = _tc_lstm1(*split_l1(l1b_Wih), col_of(l1b_bih + l1b_bhh), xT, h1T, h2T)

    hf2 = _tc_lstm2(l2f_Wih[:, :64], l2f_Wih[:, 64:],
                    col_of(l2f_bih + l2f_bhh), hf1, hb1)
    hb2 = _tc_lstm2(l2b_Wih[:, :64], l2b_Wih[:, 64:],
                    col_of(l2b_bih + l2b_bhh), hf1, hb1)

    W_parts = (fc1_W[:, :64], fc1_W[:, 64:128], fc1_W[:, 128:192],
               fc1_W[:, 192:256], fc1_W[:, 256:384])
    acts = (hf1, hb1, hf2, hb2, xT)
    return _tc_fc(W_parts, col_of(fc1_b), fc2_W, col_of(fc2_b), acts)


# speedup vs baseline: 11.2560x; 1.1092x over previous
import dataclasses
import functools

import jax
import jax.numpy as jnp
from jax import lax
from jax.experimental import pallas as pl
from jax.experimental.pallas import tpu as pltpu
from jax.experimental.pallas import tpu_sc as plsc

_N = 10000
_E = 320000
_NW = 32
_EPW = _E // _NW
_EC = 2000
_NCH = _E // _EC
_HID = 64

_HIGH = lax.Precision.HIGHEST


def _sc_mesh():
    return plsc.VectorSubcoreMesh(core_axis_name="c", subcore_axis_name="s",
                                  num_cores=2, num_subcores=16)


def _sc_params():
    cp = pltpu.CompilerParams()
    if "needs_layout_passes" in pltpu.CompilerParams.__dataclass_fields__:
        cp = dataclasses.replace(cp, needs_layout_passes=False)
    return cp


def _sc_deg(row, w):
    @functools.partial(
        pl.kernel,
        out_type=jax.ShapeDtypeStruct((_NW * _N,), jnp.float32),
        mesh=_sc_mesh(),
        compiler_params=_sc_params(),
        scratch_types=[
            pltpu.VMEM((_EPW,), jnp.int32),
            pltpu.VMEM((_EPW,), jnp.float32),
            pltpu.VMEM((_N,), jnp.float32),
        ],
    )
    def k(row_hbm, w_hbm, out_hbm, row_v, w_v, deg_v):
        wid = lax.axis_index("s") * 2 + lax.axis_index("c")
        base = wid * _EPW
        zeros = jnp.zeros((16,), jnp.float32)

        @pl.loop(0, _N, step=16)
        def _(i):
            deg_v[pl.ds(i, 16)] = zeros

        pltpu.sync_copy(row_hbm.at[pl.ds(base, _EPW)], row_v)
        pltpu.sync_copy(w_hbm.at[pl.ds(base, _EPW)], w_v)

        @pl.loop(0, _EPW, step=16)
        def _(i):
            idx = row_v[pl.ds(i, 16)]
            plsc.addupdate_scatter(deg_v, [idx], w_v[pl.ds(i, 16)])

        pltpu.sync_copy(deg_v, out_hbm.at[pl.ds(wid * _N, _N)])

    return k(row, w).reshape(_NW, _N)


def _sc_conv(hsT, row, col, w):
    nch = (_E // 2) // _EC

    @functools.partial(
        pl.kernel,
        out_type=jax.ShapeDtypeStruct((2 * _HID * _N,), jnp.float32),
        mesh=_sc_mesh(),
        compiler_params=_sc_params(),
        scratch_types=[
            pltpu.VMEM((4 * _N,), jnp.float32),
            pltpu.VMEM((4 * _N,), jnp.float32),
            pltpu.VMEM((_EC,), jnp.int32),
            pltpu.VMEM((_EC,), jnp.int32),
            pltpu.VMEM((_EC,), jnp.int32),
            pltpu.VMEM((_EC,), jnp.int32),
            pltpu.VMEM((_EC,), jnp.float32),
            pltpu.VMEM((_EC,), jnp.float32),
            pltpu.SemaphoreType.DMA,
            pltpu.SemaphoreType.DMA,
        ],
    )
    def k(h_hbm, row_hbm, col_hbm, w_hbm, out_hbm,
          ht_v, acc_v, row_v0, row_v1, col_v0, col_v1, w_v0, w_v1, sem0, sem1):
        cid = lax.axis_index("c")
        f0 = lax.axis_index("s") * 4
        ebase = cid * (_E // 2)
        bufs = ((row_v0, col_v0, w_v0, sem0), (row_v1, col_v1, w_v1, sem1))

        pltpu.sync_copy(h_hbm.at[pl.ds(f0 * _N, 4 * _N)], ht_v)

        zeros = jnp.zeros((16,), jnp.float32)

        @pl.loop(0, 4 * _N, step=16, unroll=4)
        def _(i):
            acc_v[pl.ds(i, 16)] = zeros

        def start(ch, b):
            off = ebase + ch * _EC
            rv, cv, wv_, sem = bufs[b]
            pltpu.async_copy(row_hbm.at[pl.ds(off, _EC)], rv, sem)
            pltpu.async_copy(col_hbm.at[pl.ds(off, _EC)], cv, sem)
            pltpu.async_copy(w_hbm.at[pl.ds(off, _EC)], wv_, sem)

        def wait(b):
            rv, cv, wv_, sem = bufs[b]
            pltpu.make_async_copy(row_hbm.at[pl.ds(0, _EC)], rv, sem).wait()
            pltpu.make_async_copy(col_hbm.at[pl.ds(0, _EC)], cv, sem).wait()
            pltpu.make_async_copy(w_hbm.at[pl.ds(0, _EC)], wv_, sem).wait()

        def process(b):
            rv, cv, wv_, _sem = bufs[b]

            @pl.loop(0, _EC, step=16, unroll=4)
            def _(i):
                r = rv[pl.ds(i, 16)]
                c = cv[pl.ds(i, 16)]
                wv = wv_[pl.ds(i, 16)]
                for f in range(4):
                    if f:
                        r = r + _N
                        c = c + _N
                    g = plsc.load_gather(ht_v, [r])
                    plsc.addupdate_scatter(acc_v, [c], g * wv)

        start(0, 0)
        start(1, 1)

        @pl.loop(0, nch, step=2)
        def _(ch):
            wait(0)
            process(0)

            @pl.when(ch + 2 < nch)
            def _():
                start(ch + 2, 0)

            wait(1)
            process(1)

            @pl.when(ch + 3 < nch)
            def _():
                start(ch + 3, 1)

        pltpu.sync_copy(acc_v, out_hbm.at[pl.ds((cid * _HID + f0) * _N, 4 * _N)])

    return k(hsT.reshape(-1), row, col, w).reshape(2 * _HID, _N)


def _dotT(a, b):
    return lax.dot_general(a, b, (((0,), (0,)), ((), ())),
                           preferred_element_type=jnp.float32,
                           precision=_HIGH)


def _mm(w, b):
    return lax.dot_general(w, b, (((1,), (0,)), ((), ())),
                           preferred_element_type=jnp.float32,
                           precision=_HIGH)


def _tc_pre(x, conv1_W, deg_parts):
    def body(x_ref, w_ref, dp_ref, xT_ref, h1s_ref, dinv_ref):
        xT = x_ref[...].T
        xT_ref[...] = xT
        deg = jnp.sum(dp_ref[...], axis=0, keepdims=True) + 1.0
        dinv = jnp.where(deg > 0, lax.rsqrt(jnp.maximum(deg, 1e-12)), 0.0)
        dinv_ref[...] = dinv
        h1s_ref[...] = _dotT(w_ref[...], xT) * dinv

    return pl.pallas_call(
        body,
        out_shape=(
            jax.ShapeDtypeStruct((128, _N), jnp.float32),
            jax.ShapeDtypeStruct((_HID, _N), jnp.float32),
            jax.ShapeDtypeStruct((1, _N), jnp.float32),
        ),
    )(x, conv1_W, deg_parts)


def _bn_T(h, g_col, b_col):
    m = jnp.mean(h, axis=1, keepdims=True)
    v = jnp.mean((h - m) ** 2, axis=1, keepdims=True)
    return (h - m) / jnp.sqrt(v + 1e-5) * g_col + b_col


def _tc_mid(acc1, h1s, dinv, b1_col, g1_col, bb1_col, conv2_W):
    def body(acc_ref, h1s_ref, dinv_ref, b1_ref, g1_ref, bb1_ref, w2_ref,
             h1T_ref, h2s_ref):
        dinv = dinv_ref[...]
        acc = acc_ref[0:_HID] + acc_ref[_HID:2 * _HID]
        pre = dinv * (acc + h1s_ref[...]) + b1_ref[...]
        h1 = _bn_T(jax.nn.relu(pre), g1_ref[...], bb1_ref[...])
        h1T_ref[...] = h1
        h2s_ref[...] = _dotT(w2_ref[...], h1) * dinv

    return pl.pallas_call(
        body,
        out_shape=(
            jax.ShapeDtypeStruct((_HID, _N), jnp.float32),
            jax.ShapeDtypeStruct((_HID, _N), jnp.float32),
        ),
    )(acc1, h1s, dinv, b1_col, g1_col, bb1_col, conv2_W)


def _tc_post(acc2, h2s, dinv, b2_col, g2_col, bb2_col):
    def body(acc_ref, h2s_ref, dinv_ref, b2_ref, g2_ref, bb2_ref, h2T_ref):
        acc = acc_ref[0:_HID] + acc_ref[_HID:2 * _HID]
        pre = dinv_ref[...] * (acc + h2s_ref[...]) + b2_ref[...]
        h2T_ref[...] = _bn_T(jax.nn.relu(pre), g2_ref[...], bb2_ref[...])

    return pl.pallas_call(
        body,
        out_shape=jax.ShapeDtypeStruct((_HID, _N), jnp.float32),
    )(acc2, h2s, dinv, b2_col, g2_col, bb2_col)


def _lstm_gates(gates):
    i = gates[0:_HID]
    f = gates[_HID:2 * _HID]
    g = gates[2 * _HID:3 * _HID]
    o = gates[3 * _HID:4 * _HID]
    del f
    c = jax.nn.sigmoid(i) * jnp.tanh(g)
    return jax.nn.sigmoid(o) * jnp.tanh(c)


def _tc_lstm1(Wx, Wh1, Wh2, b_col, xT, h1T, h2T):
    def body(wx_ref, w1_ref, w2_ref, b_ref, xT_ref, h1_ref, h2_ref, o_ref):
        gates = (_mm(wx_ref[...], xT_ref[...])
                 + _mm(w1_ref[...], h1_ref[...])
                 + _mm(w2_ref[...], h2_ref[...])
                 + b_ref[...])
        o_ref[...] = _lstm_gates(gates)

    return pl.pallas_call(
        body,
        out_shape=jax.ShapeDtypeStruct((_HID, _N), jnp.float32),
    )(Wx, Wh1, Wh2, b_col, xT, h1T, h2T)


def _tc_lstm2(Wa, Wb, b_col, hfT, hbT):
    def body(wa_ref, wb_ref, b_ref, hf_ref, hb_ref, o_ref):
        gates = (_mm(wa_ref[...], hf_ref[...])
                 + _mm(wb_ref[...], hb_ref[...])
                 + b_ref[...])
        o_ref[...] = _lstm_gates(gates)

    return pl.pallas_call(
        body,
        out_shape=jax.ShapeDtypeStruct((_HID, _N), jnp.float32),
    )(Wa, Wb, b_col, hfT, hbT)


def _tc_fc(W_parts, b1_col, fc2_W, b2_col, acts):
    def body(*refs):
        w_refs = refs[:5]
        b1_ref, w2_ref, b2_ref = refs[5:8]
        act_refs = refs[8:13]
        o_ref = refs[13]
        z = b1_ref[...]
        for w_r, a_r in zip(w_refs, act_refs):
            z = z + _mm(w_r[...], a_r[...])
        z = jax.nn.relu(z)
        y = _mm(w2_ref[...], z) + b2_ref[...]
        m = jnp.max(y, axis=0, keepdims=True)
        ls = y - m - jnp.log(jnp.sum(jnp.exp(y - m), axis=0, keepdims=True))
        o_ref[...] = ls.T

    return pl.pallas_call(
        body,
        out_shape=jax.ShapeDtypeStruct((_N, 10), jnp.float32),
    )(*W_parts, b1_col, fc2_W, b2_col, *acts)


def kernel(x, edge_index, edge_attr, conv1_W, conv1_b, conv2_W, conv2_b,
           bn1_g, bn1_b, bn2_g, bn2_b,
           l1f_Wih, l1f_bih, l1f_bhh, l1b_Wih, l1b_bih, l1b_bhh,
           l2f_Wih, l2f_bih, l2f_bhh, l2b_Wih, l2b_bih, l2b_bhh,
           fc1_W, fc1_b, fc2_W, fc2_b):
    row = edge_index[0]
    col = edge_index[1]

    col_of = lambda v: v.reshape(-1, 1)

    deg_parts = _sc_deg(row, edge_attr)
    xT, h1sT, dinv = _tc_pre(x, conv1_W, deg_parts)

    acc1 = _sc_conv(h1sT, row, col, edge_attr)
    h1T, h2sT = _tc_mid(acc1, h1sT, dinv, col_of(conv1_b), col_of(bn1_g),
                        col_of(bn1_b), conv2_W)

    acc2 = _sc_conv(h2sT, row, col, edge_attr)
    h2T = _tc_post(acc2, h2sT, dinv, col_of(conv2_b), col_of(bn2_g),
                   col_of(bn2_b))

    def split_l1(W):
        return W[:, :128], W[:, 128:192], W[:, 192:256]

    hf1 = _tc_lstm1(*split_l1(l1f_Wih), col_of(l1f_bih + l1f_bhh), xT, h1T, h2T)
    hb1 = _tc_lstm1(*split_l1(l1b_Wih), col_of(l1b_bih + l1b_bhh), xT, h1T, h2T)

    hf2 = _tc_lstm2(l2f_Wih[:, :64], l2f_Wih[:, 64:],
                    col_of(l2f_bih + l2f_bhh), hf1, hb1)
    hb2 = _tc_lstm2(l2b_Wih[:, :64], l2b_Wih[:, 64:],
                    col_of(l2b_bih + l2b_bhh), hf1, hb1)

    W_parts = (fc1_W[:, :64], fc1_W[:, 64:128], fc1_W[:, 128:192],
               fc1_W[:, 192:256], fc1_W[:, 256:384])
    acts = (hf1, hb1, hf2, hb2, xT)
    return _tc_fc(W_parts, col_of(fc1_b), fc2_W, col_of(fc2_b), acts)


# speedup vs baseline: 20.9921x; 1.8650x over previous
import dataclasses
import functools

import jax
import jax.numpy as jnp
from jax import lax
from jax.experimental import pallas as pl
from jax.experimental.pallas import tpu as pltpu
from jax.experimental.pallas import tpu_sc as plsc

_N = 10000
_E = 320000
_NW = 32
_EPW = _E // _NW
_EC = 2000
_NCH = _E // _EC
_HID = 64

_HIGH = lax.Precision.HIGHEST


def _sc_mesh():
    return plsc.VectorSubcoreMesh(core_axis_name="c", subcore_axis_name="s",
                                  num_cores=2, num_subcores=16)


def _sc_params():
    cp = pltpu.CompilerParams()
    if "needs_layout_passes" in pltpu.CompilerParams.__dataclass_fields__:
        cp = dataclasses.replace(cp, needs_layout_passes=False)
    return cp


def _sc_deg(row, w):
    @functools.partial(
        pl.kernel,
        out_type=jax.ShapeDtypeStruct((_NW * _N,), jnp.float32),
        mesh=_sc_mesh(),
        compiler_params=_sc_params(),
        scratch_types=[
            pltpu.VMEM((_EPW,), jnp.int32),
            pltpu.VMEM((_EPW,), jnp.float32),
            pltpu.VMEM((_N,), jnp.float32),
        ],
    )
    def k(row_hbm, w_hbm, out_hbm, row_v, w_v, deg_v):
        wid = lax.axis_index("s") * 2 + lax.axis_index("c")
        base = wid * _EPW
        zeros = jnp.zeros((16,), jnp.float32)

        @pl.loop(0, _N, step=16)
        def _(i):
            deg_v[pl.ds(i, 16)] = zeros

        pltpu.sync_copy(row_hbm.at[pl.ds(base, _EPW)], row_v)
        pltpu.sync_copy(w_hbm.at[pl.ds(base, _EPW)], w_v)

        @pl.loop(0, _EPW, step=16)
        def _(i):
            idx = row_v[pl.ds(i, 16)]
            plsc.addupdate_scatter(deg_v, [idx], w_v[pl.ds(i, 16)])

        pltpu.sync_copy(deg_v, out_hbm.at[pl.ds(wid * _N, _N)])

    return k(row, w).reshape(_NW, _N)


def _sc_conv(hsT, row, col, w):
    nch = (_E // 2) // _EC

    @functools.partial(
        pl.kernel,
        out_type=jax.ShapeDtypeStruct((2 * _HID * _N,), jnp.float32),
        mesh=_sc_mesh(),
        compiler_params=_sc_params(),
        scratch_types=[
            pltpu.VMEM((4 * _N,), jnp.float32),
            pltpu.VMEM((4 * _N,), jnp.float32),
            pltpu.VMEM((_EC,), jnp.int32),
            pltpu.VMEM((_EC,), jnp.int32),
            pltpu.VMEM((_EC,), jnp.int32),
            pltpu.VMEM((_EC,), jnp.int32),
            pltpu.VMEM((_EC,), jnp.float32),
            pltpu.VMEM((_EC,), jnp.float32),
            pltpu.SemaphoreType.DMA,
            pltpu.SemaphoreType.DMA,
        ],
    )
    def k(h_hbm, row_hbm, col_hbm, w_hbm, out_hbm,
          ht_v, acc_v, row_v0, row_v1, col_v0, col_v1, w_v0, w_v1, sem0, sem1):
        cid = lax.axis_index("c")
        f0 = lax.axis_index("s") * 4
        ebase = cid * (_E // 2)
        bufs = ((row_v0, col_v0, w_v0, sem0), (row_v1, col_v1, w_v1, sem1))

        pltpu.sync_copy(h_hbm.at[pl.ds(f0 * _N, 4 * _N)], ht_v)

        zeros = jnp.zeros((16,), jnp.float32)

        @pl.loop(0, 4 * _N, step=16, unroll=4)
        def _(i):
            acc_v[pl.ds(i, 16)] = zeros

        def start(ch, b):
            off = ebase + ch * _EC
            rv, cv, wv_, sem = bufs[b]
            pltpu.async_copy(row_hbm.at[pl.ds(off, _EC)], rv, sem)
            pltpu.async_copy(col_hbm.at[pl.ds(off, _EC)], cv, sem)
            pltpu.async_copy(w_hbm.at[pl.ds(off, _EC)], wv_, sem)

        def wait(b):
            rv, cv, wv_, sem = bufs[b]
            pltpu.make_async_copy(row_hbm.at[pl.ds(0, _EC)], rv, sem).wait()
            pltpu.make_async_copy(col_hbm.at[pl.ds(0, _EC)], cv, sem).wait()
            pltpu.make_async_copy(w_hbm.at[pl.ds(0, _EC)], wv_, sem).wait()

        def process(b):
            rv, cv, wv_, _sem = bufs[b]

            @plsc.parallel_loop(0, _EC, step=16, unroll=4)
            def _(i):
                r = rv[pl.ds(i, 16)]
                c = cv[pl.ds(i, 16)]
                wv = wv_[pl.ds(i, 16)]
                for f in range(4):
                    if f:
                        r = r + _N
                        c = c + _N
                    g = plsc.load_gather(ht_v, [r])
                    plsc.addupdate_scatter(acc_v, [c], g * wv)

        start(0, 0)
        start(1, 1)

        @pl.loop(0, nch, step=2)
        def _(ch):
            wait(0)
            process(0)

            @pl.when(ch + 2 < nch)
            def _():
                start(ch + 2, 0)

            wait(1)
            process(1)

            @pl.when(ch + 3 < nch)
            def _():
                start(ch + 3, 1)

        pltpu.sync_copy(acc_v, out_hbm.at[pl.ds((cid * _HID + f0) * _N, 4 * _N)])

    return k(hsT.reshape(-1), row, col, w).reshape(2 * _HID, _N)


def _dotT(a, b):
    return lax.dot_general(a, b, (((0,), (0,)), ((), ())),
                           preferred_element_type=jnp.float32,
                           precision=_HIGH)


def _mm(w, b):
    return lax.dot_general(w, b, (((1,), (0,)), ((), ())),
                           preferred_element_type=jnp.float32,
                           precision=_HIGH)


def _tc_pre(x, conv1_W, deg_parts):
    def body(x_ref, w_ref, dp_ref, xT_ref, h1s_ref, dinv_ref):
        xT = x_ref[...].T
        xT_ref[...] = xT
        deg = jnp.sum(dp_ref[...], axis=0, keepdims=True) + 1.0
        dinv = jnp.where(deg > 0, lax.rsqrt(jnp.maximum(deg, 1e-12)), 0.0)
        dinv_ref[...] = dinv
        h1s_ref[...] = _dotT(w_ref[...], xT) * dinv

    return pl.pallas_call(
        body,
        out_shape=(
            jax.ShapeDtypeStruct((128, _N), jnp.float32),
            jax.ShapeDtypeStruct((_HID, _N), jnp.float32),
            jax.ShapeDtypeStruct((1, _N), jnp.float32),
        ),
    )(x, conv1_W, deg_parts)


def _bn_T(h, g_col, b_col):
    m = jnp.mean(h, axis=1, keepdims=True)
    v = jnp.mean((h - m) ** 2, axis=1, keepdims=True)
    return (h - m) / jnp.sqrt(v + 1e-5) * g_col + b_col


def _tc_mid(acc1, h1s, dinv, b1_col, g1_col, bb1_col, conv2_W):
    def body(acc_ref, h1s_ref, dinv_ref, b1_ref, g1_ref, bb1_ref, w2_ref,
             h1T_ref, h2s_ref):
        dinv = dinv_ref[...]
        acc = acc_ref[0:_HID] + acc_ref[_HID:2 * _HID]
        pre = dinv * (acc + h1s_ref[...]) + b1_ref[...]
        h1 = _bn_T(jax.nn.relu(pre), g1_ref[...], bb1_ref[...])
        h1T_ref[...] = h1
        h2s_ref[...] = _dotT(w2_ref[...], h1) * dinv

    return pl.pallas_call(
        body,
        out_shape=(
            jax.ShapeDtypeStruct((_HID, _N), jnp.float32),
            jax.ShapeDtypeStruct((_HID, _N), jnp.float32),
        ),
    )(acc1, h1s, dinv, b1_col, g1_col, bb1_col, conv2_W)


def _tc_post(acc2, h2s, dinv, b2_col, g2_col, bb2_col):
    def body(acc_ref, h2s_ref, dinv_ref, b2_ref, g2_ref, bb2_ref, h2T_ref):
        acc = acc_ref[0:_HID] + acc_ref[_HID:2 * _HID]
        pre = dinv_ref[...] * (acc + h2s_ref[...]) + b2_ref[...]
        h2T_ref[...] = _bn_T(jax.nn.relu(pre), g2_ref[...], bb2_ref[...])

    return pl.pallas_call(
        body,
        out_shape=jax.ShapeDtypeStruct((_HID, _N), jnp.float32),
    )(acc2, h2s, dinv, b2_col, g2_col, bb2_col)


def _lstm_gates(gates):
    i = gates[0:_HID]
    f = gates[_HID:2 * _HID]
    g = gates[2 * _HID:3 * _HID]
    o = gates[3 * _HID:4 * _HID]
    del f
    c = jax.nn.sigmoid(i) * jnp.tanh(g)
    return jax.nn.sigmoid(o) * jnp.tanh(c)


def _tc_lstm1(Wx, Wh1, Wh2, b_col, xT, h1T, h2T):
    def body(wx_ref, w1_ref, w2_ref, b_ref, xT_ref, h1_ref, h2_ref, o_ref):
        gates = (_mm(wx_ref[...], xT_ref[...])
                 + _mm(w1_ref[...], h1_ref[...])
                 + _mm(w2_ref[...], h2_ref[...])
                 + b_ref[...])
        o_ref[...] = _lstm_gates(gates)

    return pl.pallas_call(
        body,
        out_shape=jax.ShapeDtypeStruct((_HID, _N), jnp.float32),
    )(Wx, Wh1, Wh2, b_col, xT, h1T, h2T)


def _tc_lstm2(Wa, Wb, b_col, hfT, hbT):
    def body(wa_ref, wb_ref, b_ref, hf_ref, hb_ref, o_ref):
        gates = (_mm(wa_ref[...], hf_ref[...])
                 + _mm(wb_ref[...], hb_ref[...])
                 + b_ref[...])
        o_ref[...] = _lstm_gates(gates)

    return pl.pallas_call(
        body,
        out_shape=jax.ShapeDtypeStruct((_HID, _N), jnp.float32),
    )(Wa, Wb, b_col, hfT, hbT)


def _tc_fc(W_parts, b1_col, fc2_W, b2_col, acts):
    def body(*refs):
        w_refs = refs[:5]
        b1_ref, w2_ref, b2_ref = refs[5:8]
        act_refs = refs[8:13]
        o_ref = refs[13]
        z = b1_ref[...]
        for w_r, a_r in zip(w_refs, act_refs):
            z = z + _mm(w_r[...], a_r[...])
        z = jax.nn.relu(z)
        y = _mm(w2_ref[...], z) + b2_ref[...]
        m = jnp.max(y, axis=0, keepdims=True)
        ls = y - m - jnp.log(jnp.sum(jnp.exp(y - m), axis=0, keepdims=True))
        o_ref[...] = ls.T

    return pl.pallas_call(
        body,
        out_shape=jax.ShapeDtypeStruct((_N, 10), jnp.float32),
    )(*W_parts, b1_col, fc2_W, b2_col, *acts)


def kernel(x, edge_index, edge_attr, conv1_W, conv1_b, conv2_W, conv2_b,
           bn1_g, bn1_b, bn2_g, bn2_b,
           l1f_Wih, l1f_bih, l1f_bhh, l1b_Wih, l1b_bih, l1b_bhh,
           l2f_Wih, l2f_bih, l2f_bhh, l2b_Wih, l2b_bih, l2b_bhh,
           fc1_W, fc1_b, fc2_W, fc2_b):
    row = edge_index[0]
    col = edge_index[1]

    col_of = lambda v: v.reshape(-1, 1)

    deg_parts = _sc_deg(row, edge_attr)
    xT, h1sT, dinv = _tc_pre(x, conv1_W, deg_parts)

    acc1 = _sc_conv(h1sT, row, col, edge_attr)
    h1T, h2sT = _tc_mid(acc1, h1sT, dinv, col_of(conv1_b), col_of(bn1_g),
                        col_of(bn1_b), conv2_W)

    acc2 = _sc_conv(h2sT, row, col, edge_attr)
    h2T = _tc_post(acc2, h2sT, dinv, col_of(conv2_b), col_of(bn2_g),
                   col_of(bn2_b))

    def split_l1(W):
        return W[:, :128], W[:, 128:192], W[:, 192:256]

    hf1 = _tc_lstm1(*split_l1(l1f_Wih), col_of(l1f_bih + l1f_bhh), xT, h1T, h2T)
    hb1 = _tc_lstm1(*split_l1(l1b_Wih), col_of(l1b_bih + l1b_bhh), xT, h1T, h2T)

    hf2 = _tc_lstm2(l2f_Wih[:, :64], l2f_Wih[:, 64:],
                    col_of(l2f_bih + l2f_bhh), hf1, hb1)
    hb2 = _tc_lstm2(l2b_Wih[:, :64], l2b_Wih[:, 64:],
                    col_of(l2b_bih + l2b_bhh), hf1, hb1)

    W_parts = (fc1_W[:, :64], fc1_W[:, 64:128], fc1_W[:, 128:192],
               fc1_W[:, 192:256], fc1_W[:, 256:384])
    acts = (hf1, hb1, hf2, hb2, xT)
    return _tc_fc(W_parts, col_of(fc1_b), fc2_W, col_of(fc2_b), acts)


# speedup vs baseline: 23.3569x; 1.1126x over previous
import dataclasses
import functools

import jax
import jax.numpy as jnp
from jax import lax
from jax.experimental import pallas as pl
from jax.experimental.pallas import tpu as pltpu
from jax.experimental.pallas import tpu_sc as plsc

_N = 10000
_E = 320000
_NW = 32
_EPW = _E // _NW
_EC = 2000
_NCH = _E // _EC
_HID = 64

_HIGH = lax.Precision.HIGHEST


def _sc_mesh():
    return plsc.VectorSubcoreMesh(core_axis_name="c", subcore_axis_name="s",
                                  num_cores=2, num_subcores=16)


def _sc_params():
    cp = pltpu.CompilerParams()
    if "needs_layout_passes" in pltpu.CompilerParams.__dataclass_fields__:
        cp = dataclasses.replace(cp, needs_layout_passes=False)
    return cp


def _sc_deg(row, w):
    @functools.partial(
        pl.kernel,
        out_type=jax.ShapeDtypeStruct((_NW * _N,), jnp.float32),
        mesh=_sc_mesh(),
        compiler_params=_sc_params(),
        scratch_types=[
            pltpu.VMEM((_EPW,), jnp.int32),
            pltpu.VMEM((_EPW,), jnp.float32),
            pltpu.VMEM((_N,), jnp.float32),
        ],
    )
    def k(row_hbm, w_hbm, out_hbm, row_v, w_v, deg_v):
        wid = lax.axis_index("s") * 2 + lax.axis_index("c")
        base = wid * _EPW
        zeros = jnp.zeros((16,), jnp.float32)

        @pl.loop(0, _N, step=16)
        def _(i):
            deg_v[pl.ds(i, 16)] = zeros

        pltpu.sync_copy(row_hbm.at[pl.ds(base, _EPW)], row_v)
        pltpu.sync_copy(w_hbm.at[pl.ds(base, _EPW)], w_v)

        @pl.loop(0, _EPW, step=16)
        def _(i):
            idx = row_v[pl.ds(i, 16)]
            plsc.addupdate_scatter(deg_v, [idx], w_v[pl.ds(i, 16)])

        pltpu.sync_copy(deg_v, out_hbm.at[pl.ds(wid * _N, _N)])

    return k(row, w).reshape(_NW, _N)


def _sc_conv(hsT, row, col, w):
    nch = (_E // 2) // _EC

    @functools.partial(
        pl.kernel,
        out_type=jax.ShapeDtypeStruct((2 * _HID * _N,), jnp.float32),
        mesh=_sc_mesh(),
        compiler_params=_sc_params(),
        scratch_types=[
            pltpu.VMEM((4 * _N,), jnp.float32),
            pltpu.VMEM((4 * _N,), jnp.float32),
            pltpu.VMEM((_EC,), jnp.int32),
            pltpu.VMEM((_EC,), jnp.int32),
            pltpu.VMEM((_EC,), jnp.int32),
            pltpu.VMEM((_EC,), jnp.int32),
            pltpu.VMEM((_EC,), jnp.float32),
            pltpu.VMEM((_EC,), jnp.float32),
            pltpu.SemaphoreType.DMA,
            pltpu.SemaphoreType.DMA,
        ],
    )
    def k(h_hbm, row_hbm, col_hbm, w_hbm, out_hbm,
          ht_v, acc_v, row_v0, row_v1, col_v0, col_v1, w_v0, w_v1, sem0, sem1):
        cid = lax.axis_index("c")
        f0 = lax.axis_index("s") * 4
        ebase = cid * (_E // 2)
        bufs = ((row_v0, col_v0, w_v0, sem0), (row_v1, col_v1, w_v1, sem1))

        pltpu.sync_copy(h_hbm.at[pl.ds(f0 * _N, 4 * _N)], ht_v)

        zeros = jnp.zeros((16,), jnp.float32)

        @pl.loop(0, 4 * _N, step=16, unroll=4)
        def _(i):
            acc_v[pl.ds(i, 16)] = zeros

        def start(ch, b):
            off = ebase + ch * _EC
            rv, cv, wv_, sem = bufs[b]
            pltpu.async_copy(row_hbm.at[pl.ds(off, _EC)], rv, sem)
            pltpu.async_copy(col_hbm.at[pl.ds(off, _EC)], cv, sem)
            pltpu.async_copy(w_hbm.at[pl.ds(off, _EC)], wv_, sem)

        def wait(b):
            rv, cv, wv_, sem = bufs[b]
            pltpu.make_async_copy(row_hbm.at[pl.ds(0, _EC)], rv, sem).wait()
            pltpu.make_async_copy(col_hbm.at[pl.ds(0, _EC)], cv, sem).wait()
            pltpu.make_async_copy(w_hbm.at[pl.ds(0, _EC)], wv_, sem).wait()

        def process(b):
            rv, cv, wv_, _sem = bufs[b]

            @plsc.parallel_loop(0, _EC, step=16, unroll=4)
            def _(i):
                r = rv[pl.ds(i, 16)]
                c = cv[pl.ds(i, 16)]
                wv = wv_[pl.ds(i, 16)]
                for f in range(4):
                    if f:
                        r = r + _N
                        c = c + _N
                    g = plsc.load_gather(ht_v, [r])
                    plsc.addupdate_scatter(acc_v, [c], g * wv)

        start(0, 0)
        start(1, 1)

        @pl.loop(0, nch, step=2)
        def _(ch):
            wait(0)
            process(0)

            @pl.when(ch + 2 < nch)
            def _():
                start(ch + 2, 0)

            wait(1)
            process(1)

            @pl.when(ch + 3 < nch)
            def _():
                start(ch + 3, 1)

        pltpu.sync_copy(acc_v, out_hbm.at[pl.ds((cid * _HID + f0) * _N, 4 * _N)])

    return k(hsT.reshape(-1), row, col, w).reshape(2 * _HID, _N)


def _dotT(a, b):
    return lax.dot_general(a, b, (((0,), (0,)), ((), ())),
                           preferred_element_type=jnp.float32,
                           precision=_HIGH)


def _mm(w, b):
    return lax.dot_general(w, b, (((1,), (0,)), ((), ())),
                           preferred_element_type=jnp.float32,
                           precision=_HIGH)


def _tc_pre(x, conv1_W, deg_parts):
    def body(x_ref, w_ref, dp_ref, xT_ref, h1s_ref, dinv_ref):
        xT = x_ref[...].T
        xT_ref[...] = xT
        deg = jnp.sum(dp_ref[...], axis=0, keepdims=True) + 1.0
        dinv = jnp.where(deg > 0, lax.rsqrt(jnp.maximum(deg, 1e-12)), 0.0)
        dinv_ref[...] = dinv
        h1s_ref[...] = _dotT(w_ref[...], xT) * dinv

    return pl.pallas_call(
        body,
        out_shape=(
            jax.ShapeDtypeStruct((128, _N), jnp.float32),
            jax.ShapeDtypeStruct((_HID, _N), jnp.float32),
            jax.ShapeDtypeStruct((1, _N), jnp.float32),
        ),
    )(x, conv1_W, deg_parts)


def _bn_T(h, g_col, b_col):
    m = jnp.mean(h, axis=1, keepdims=True)
    v = jnp.mean((h - m) ** 2, axis=1, keepdims=True)
    return (h - m) / jnp.sqrt(v + 1e-5) * g_col + b_col


def _tc_mid(acc1, h1s, dinv, b1_col, g1_col, bb1_col, conv2_W):
    def body(acc_ref, h1s_ref, dinv_ref, b1_ref, g1_ref, bb1_ref, w2_ref,
             h1T_ref, h2s_ref):
        dinv = dinv_ref[...]
        acc = acc_ref[0:_HID] + acc_ref[_HID:2 * _HID]
        pre = dinv * (acc + h1s_ref[...]) + b1_ref[...]
        h1 = _bn_T(jax.nn.relu(pre), g1_ref[...], bb1_ref[...])
        h1T_ref[...] = h1
        h2s_ref[...] = _dotT(w2_ref[...], h1) * dinv

    return pl.pallas_call(
        body,
        out_shape=(
            jax.ShapeDtypeStruct((_HID, _N), jnp.float32),
            jax.ShapeDtypeStruct((_HID, _N), jnp.float32),
        ),
    )(acc1, h1s, dinv, b1_col, g1_col, bb1_col, conv2_W)


def _tc_post(acc2, h2s, dinv, b2_col, g2_col, bb2_col):
    def body(acc_ref, h2s_ref, dinv_ref, b2_ref, g2_ref, bb2_ref, h2T_ref):
        acc = acc_ref[0:_HID] + acc_ref[_HID:2 * _HID]
        pre = dinv_ref[...] * (acc + h2s_ref[...]) + b2_ref[...]
        h2T_ref[...] = _bn_T(jax.nn.relu(pre), g2_ref[...], bb2_ref[...])

    return pl.pallas_call(
        body,
        out_shape=jax.ShapeDtypeStruct((_HID, _N), jnp.float32),
    )(acc2, h2s, dinv, b2_col, g2_col, bb2_col)


def _lstm_gates(gates):
    i = gates[0:_HID]
    f = gates[_HID:2 * _HID]
    g = gates[2 * _HID:3 * _HID]
    o = gates[3 * _HID:4 * _HID]
    del f
    c = jax.nn.sigmoid(i) * jnp.tanh(g)
    return jax.nn.sigmoid(o) * jnp.tanh(c)


def _tc_post(acc2, h2s, dinv, b2_col, g2_col, bb2_col):
    def body(acc_ref, h2s_ref, dinv_ref, b2_ref, g2_ref, bb2_ref, h2T_ref):
        acc = acc_ref[0:_HID] + acc_ref[_HID:2 * _HID]
        pre = dinv_ref[...] * (acc + h2s_ref[...]) + b2_ref[...]
        h2T_ref[...] = _bn_T(jax.nn.relu(pre), g2_ref[...], bb2_ref[...])

    return pl.pallas_call(
        body,
        out_shape=jax.ShapeDtypeStruct((_HID, _N), jnp.float32),
    )(acc2, h2s, dinv, b2_col, g2_col, bb2_col)


def _tc_gates_pre(Wxf, Wh1f, bf_col, Wxb, Wh1b, bb_col, Wfc_x, bfc_col,
                  xT, h1T):
    def body(wxf_ref, w1f_ref, bf_ref, wxb_ref, w1b_ref, bb_ref, wfc_ref,
             bfc_ref, xT_ref, h1_ref, pf_ref, pb_ref, pfc_ref):
        xT = xT_ref[...]
        h1 = h1_ref[...]
        pf_ref[...] = _mm(wxf_ref[...], xT) + _mm(w1f_ref[...], h1) + bf_ref[...]
        pb_ref[...] = _mm(wxb_ref[...], xT) + _mm(w1b_ref[...], h1) + bb_ref[...]
        pfc_ref[...] = _mm(wfc_ref[...], xT) + bfc_ref[...]

    return pl.pallas_call(
        body,
        out_shape=(
            jax.ShapeDtypeStruct((4 * _HID, _N), jnp.float32),
            jax.ShapeDtypeStruct((4 * _HID, _N), jnp.float32),
            jax.ShapeDtypeStruct((_HID, _N), jnp.float32),
        ),
    )(Wxf, Wh1f, bf_col, Wxb, Wh1b, bb_col, Wfc_x, bfc_col, xT, h1T)


def _lstm_gates(gates):
    i = gates[0:_HID]
    g = gates[2 * _HID:3 * _HID]
    o = gates[3 * _HID:4 * _HID]
    c = jax.nn.sigmoid(i) * jnp.tanh(g)
    return jax.nn.sigmoid(o) * jnp.tanh(c)


def _tc_head(h2T,
             Wh2f, Wh2b, l2f_a, l2f_b, b2f_col, l2b_a, l2b_b, b2b_col,
             Wz_f1, Wz_b1, Wz_f2, Wz_b2, fc2_W, bfc2_col,
             P1f, P1b, Pfc):
    def body(h2_ref,
             wh2f_ref, wh2b_ref, l2fa_ref, l2fb_ref, b2f_ref,
             l2ba_ref, l2bb_ref, b2b_ref,
             wzf1_ref, wzb1_ref, wzf2_ref, wzb2_ref, w2_ref, bfc2_ref,
             pf_ref, pb_ref, pfc_ref, o_ref):
        h2 = h2_ref[...]
        hf1 = _lstm_gates(pf_ref[...] + _mm(wh2f_ref[...], h2))
        hb1 = _lstm_gates(pb_ref[...] + _mm(wh2b_ref[...], h2))
        hf2 = _lstm_gates(_mm(l2fa_ref[...], hf1) + _mm(l2fb_ref[...], hb1)
                          + b2f_ref[...])
        hb2 = _lstm_gates(_mm(l2ba_ref[...], hf1) + _mm(l2bb_ref[...], hb1)
                          + b2b_ref[...])
        z = jax.nn.relu(pfc_ref[...] + _mm(wzf1_ref[...], hf1)
                        + _mm(wzb1_ref[...], hb1) + _mm(wzf2_ref[...], hf2)
                        + _mm(wzb2_ref[...], hb2))
        y = _mm(w2_ref[...], z) + bfc2_ref[...]
        m = jnp.max(y, axis=0, keepdims=True)
        ls = y - m - jnp.log(jnp.sum(jnp.exp(y - m), axis=0, keepdims=True))
        o_ref[...] = ls.T

    return pl.pallas_call(
        body,
        out_shape=jax.ShapeDtypeStruct((_N, 10), jnp.float32),
    )(h2T,
      Wh2f, Wh2b, l2f_a, l2f_b, b2f_col, l2b_a, l2b_b, b2b_col,
      Wz_f1, Wz_b1, Wz_f2, Wz_b2, fc2_W, bfc2_col, P1f, P1b, Pfc)


def kernel(x, edge_index, edge_attr, conv1_W, conv1_b, conv2_W, conv2_b,
           bn1_g, bn1_b, bn2_g, bn2_b,
           l1f_Wih, l1f_bih, l1f_bhh, l1b_Wih, l1b_bih, l1b_bhh,
           l2f_Wih, l2f_bih, l2f_bhh, l2b_Wih, l2b_bih, l2b_bhh,
           fc1_W, fc1_b, fc2_W, fc2_b):
    row = edge_index[0]
    col = edge_index[1]

    col_of = lambda v: v.reshape(-1, 1)

    deg_parts = _sc_deg(row, edge_attr)
    xT, h1sT, dinv = _tc_pre(x, conv1_W, deg_parts)

    acc1 = _sc_conv(h1sT, row, col, edge_attr)
    h1T, h2sT = _tc_mid(acc1, h1sT, dinv, col_of(conv1_b), col_of(bn1_g),
                        col_of(bn1_b), conv2_W)

    acc2 = _sc_conv(h2sT, row, col, edge_attr)

    P1f, P1b, Pfc = _tc_gates_pre(
        l1f_Wih[:, :128], l1f_Wih[:, 128:192], col_of(l1f_bih + l1f_bhh),
        l1b_Wih[:, :128], l1b_Wih[:, 128:192], col_of(l1b_bih + l1b_bhh),
        fc1_W[:, 256:384], col_of(fc1_b), xT, h1T)

    h2T = _tc_post(acc2, h2sT, dinv, col_of(conv2_b), col_of(bn2_g),
                   col_of(bn2_b))

    return _tc_head(
        h2T,
        l1f_Wih[:, 192:256], l1b_Wih[:, 192:256],
        l2f_Wih[:, :64], l2f_Wih[:, 64:], col_of(l2f_bih + l2f_bhh),
        l2b_Wih[:, :64], l2b_Wih[:, 64:], col_of(l2b_bih + l2b_bhh),
        fc1_W[:, :64], fc1_W[:, 64:128], fc1_W[:, 128:192], fc1_W[:, 192:256],
        fc2_W, col_of(fc2_b), P1f, P1b, Pfc)
